# Initial kernel scaffold; baseline (speedup 1.0000x reference)
import jax
import jax.numpy as jnp
from jax.experimental import pallas as pl


def kernel(x, edge_index, Wrel1, Wroot1, b1, Wrel2, Wroot2, b2, fcW, fcb):
    raise NotImplementedError("write your pallas kernel here")


# speedup vs baseline: 29.4592x; 29.4592x over previous
import functools

import jax
import jax.numpy as jnp
from jax import lax
from jax.experimental import pallas as pl
from jax.experimental.pallas import tpu as pltpu
from jax.experimental.pallas import tpu_sc as plsc

N = 100000
E = 3200000
F = 16

NUM_CORES = 2
NUM_SUBCORES = 16
NUM_TILES = NUM_CORES * NUM_SUBCORES

CHUNK = 128
K = 8
ROWS_PER_TILE = 784
GROUPS = ROWS_PER_TILE // K
EDGES_PER_TILE = ROWS_PER_TILE * CHUNK
E_PAD = NUM_TILES * EDGES_PER_TILE
N_ACC = 100352
T_ROWS = N_ACC // NUM_SUBCORES


def _sc_pass(with_deg: bool):
    mesh = plsc.VectorSubcoreMesh(
        core_axis_name="c", subcore_axis_name="s",
        num_cores=NUM_CORES, num_subcores=NUM_SUBCORES)

    out_type = [jax.ShapeDtypeStruct((NUM_CORES, N_ACC, F), jnp.float32)]
    scratch = [
        pltpu.VMEM((K, CHUNK), jnp.int32),
        pltpu.VMEM((K, CHUNK), jnp.int32),
        pltpu.VMEM((K, CHUNK, F), jnp.float32),
        pltpu.VMEM_SHARED((N_ACC, F), jnp.float32),
        pltpu.SemaphoreType.DMA,
    ]
    if with_deg:
        out_type.append(jax.ShapeDtypeStruct((NUM_CORES, N_ACC), jnp.float32))
        scratch += [
            pltpu.VMEM((CHUNK,), jnp.float32),
            pltpu.VMEM_SHARED((N_ACC,), jnp.float32),
        ]

    def body(table, src2d, dst2d, z2, z1, *refs):
        if with_deg:
            out, deg_out, src_v, dst_v, rows_v, acc, sem, ones_v, deg_acc = refs
        else:
            out, src_v, dst_v, rows_v, acc, sem = refs
        c = lax.axis_index("c")
        s = lax.axis_index("s")

        pltpu.sync_copy(z2.at[pl.ds(s * T_ROWS, T_ROWS)],
                        acc.at[pl.ds(s * T_ROWS, T_ROWS)])
        if with_deg:
            pltpu.sync_copy(z1.at[pl.ds(s * T_ROWS, T_ROWS)],
                            deg_acc.at[pl.ds(s * T_ROWS, T_ROWS)])
            for i in range(CHUNK // 16):
                ones_v[pl.ds(i * 16, 16)] = jnp.ones((16,), jnp.float32)
        plsc.subcore_barrier()

        tile_id = c * NUM_SUBCORES + s
        row_base = tile_id * ROWS_PER_TILE

        def group(g, carry):
            base = row_base + g * K
            pltpu.sync_copy(src2d.at[pl.ds(base, K)], src_v)
            pltpu.sync_copy(dst2d.at[pl.ds(base, K)], dst_v)
            descs = [pltpu.async_copy(table.at[src_v.at[j]], rows_v.at[j], sem)
                     for j in range(K)]
            for d in descs:
                d.wait()
            for j in range(K):
                pltpu.sync_copy(rows_v.at[j], acc.at[dst_v.at[j]], add=True)
            if with_deg:
                for j in range(K):
                    pltpu.sync_copy(ones_v, deg_acc.at[dst_v.at[j]], add=True)
            return carry

        lax.fori_loop(0, GROUPS, group, 0)
        plsc.subcore_barrier()

        pltpu.sync_copy(acc.at[pl.ds(s * T_ROWS, T_ROWS)],
                        out.at[c, pl.ds(s * T_ROWS, T_ROWS)])
        if with_deg:
            pltpu.sync_copy(deg_acc.at[pl.ds(s * T_ROWS, T_ROWS)],
                            deg_out.at[c, pl.ds(s * T_ROWS, T_ROWS)])

    return pl.kernel(
        body, out_type=out_type, mesh=mesh, scratch_types=scratch,
        compiler_params=pltpu.CompilerParams(use_tc_tiling_on_sc=False))


BLK = 3584
GRID = N_ACC // BLK


def _tc_merge(parts):
    def body(p_ref, o_ref):
        o_ref[...] = p_ref[0] + p_ref[1]
    return pl.pallas_call(
        body,
        grid=(GRID,),
        in_specs=[pl.BlockSpec((NUM_CORES, BLK, F), lambda i: (0, i, 0))],
        out_specs=pl.BlockSpec((BLK, F), lambda i: (i, 0)),
        out_shape=jax.ShapeDtypeStruct((N_ACC, F), jnp.float32),
    )(parts)


def _tc_final(x_pad, agg1, b_parts, deg_parts, Wrel1, Wroot1, b1, Wrel2,
              Wroot2, b2, fcW, fcb):
    def body(x_ref, a1_ref, bp_ref, dp_ref, wr1_ref, wo1_ref, b1_ref,
             wr2_ref, wo2_ref, b2_ref, fw_ref, fb_ref, o_ref):
        x = x_ref[...]
        a1 = a1_ref[...]
        B = bp_ref[0] + bp_ref[1]
        deg = dp_ref[0] + dp_ref[1]
        dot = functools.partial(jnp.dot, preferred_element_type=jnp.float32)
        h1 = dot(a1, wr1_ref[...].T) + b1_ref[...] + dot(x, wo1_ref[...].T)
        agg2 = (dot(B, wr1_ref[...].T) + dot(a1, wo1_ref[...].T)
                + deg * b1_ref[...])
        h2 = dot(agg2, wr2_ref[...].T) + b2_ref[...] + dot(h1, wo2_ref[...].T)
        o_ref[...] = dot(h2, fw_ref[...].T) + fb_ref[...]

    full = lambda shape: pl.BlockSpec(shape, lambda i: tuple(0 for _ in shape))
    return pl.pallas_call(
        body,
        grid=(GRID,),
        in_specs=[
            pl.BlockSpec((BLK, F), lambda i: (i, 0)),
            pl.BlockSpec((BLK, F), lambda i: (i, 0)),
            pl.BlockSpec((NUM_CORES, BLK, F), lambda i: (0, i, 0)),
            pl.BlockSpec((NUM_CORES, BLK, 1), lambda i: (0, i, 0)),
            full((32, 16)), full((32, 16)), full((32,)),
            full((64, 32)), full((64, 32)), full((64,)),
            full((10, 64)), full((10,)),
        ],
        out_specs=pl.BlockSpec((BLK, 10), lambda i: (i, 0)),
        out_shape=jax.ShapeDtypeStruct((N_ACC, 10), jnp.float32),
    )(x_pad, agg1, b_parts, deg_parts, Wrel1, Wroot1, b1, Wrel2, Wroot2, b2,
      fcW, fcb)


def kernel(x, edge_index, Wrel1, Wroot1, b1, Wrel2, Wroot2, b2, fcW, fcb):
    src = edge_index[0].astype(jnp.int32)
    dst = edge_index[1].astype(jnp.int32)
    P = E_PAD - E
    pad_i = jnp.arange(P, dtype=jnp.int32)
    src_p = jnp.concatenate([src, pad_i % 128]).reshape(E_PAD // CHUNK, CHUNK)
    dst_p = jnp.concatenate([dst, N + (pad_i % 256)]).reshape(
        E_PAD // CHUNK, CHUNK)

    zeros2 = jnp.zeros((N_ACC, F), jnp.float32)
    zeros1 = jnp.zeros((N_ACC,), jnp.float32)
    x_pad = jnp.concatenate([x, jnp.zeros((N_ACC - N, F), jnp.float32)])

    agg1_parts, deg_parts = _sc_pass(True)(x_pad, src_p, dst_p, zeros2, zeros1)
    agg1 = _tc_merge(agg1_parts)
    (b_parts,) = _sc_pass(False)(agg1, src_p, dst_p, zeros2, zeros1)
    out = _tc_final(x_pad, agg1, b_parts,
                    deg_parts.reshape(NUM_CORES, N_ACC, 1), Wrel1, Wroot1, b1,
                    Wrel2, Wroot2, b2, fcW, fcb)
    return out[:N]


# speedup vs baseline: 39.5686x; 1.3432x over previous
import functools

import jax
import jax.numpy as jnp
from jax import lax
from jax.experimental import pallas as pl
from jax.experimental.pallas import tpu as pltpu
from jax.experimental.pallas import tpu_sc as plsc

N = 100000
E = 3200000
F = 16

NUM_CORES = 2
NUM_SUBCORES = 16
NUM_TILES = NUM_CORES * NUM_SUBCORES

CHUNK = 128
K = 4
ROWS_PER_TILE = 784
GROUPS = ROWS_PER_TILE // K
EDGES_PER_TILE = ROWS_PER_TILE * CHUNK
E_PAD = NUM_TILES * EDGES_PER_TILE
N_ACC = 100352
T_ROWS = N_ACC // NUM_SUBCORES


def _sc_pass(with_deg: bool):
    mesh = plsc.VectorSubcoreMesh(
        core_axis_name="c", subcore_axis_name="s",
        num_cores=NUM_CORES, num_subcores=NUM_SUBCORES)

    out_type = [jax.ShapeDtypeStruct((NUM_CORES, N_ACC, F), jnp.float32)]
    scratch = [
        pltpu.VMEM((2, K, CHUNK), jnp.int32),
        pltpu.VMEM((2, K, CHUNK), jnp.int32),
        pltpu.VMEM((2, K, CHUNK, F), jnp.float32),
        pltpu.VMEM_SHARED((N_ACC, F), jnp.float32),
        pltpu.SemaphoreType.DMA,
        pltpu.SemaphoreType.DMA,
        pltpu.SemaphoreType.DMA,
    ]
    if with_deg:
        out_type.append(jax.ShapeDtypeStruct((NUM_CORES, N_ACC), jnp.float32))
        scratch += [
            pltpu.VMEM((CHUNK,), jnp.float32),
            pltpu.VMEM_SHARED((N_ACC,), jnp.float32),
            pltpu.SemaphoreType.DMA,
        ]

    def body(table, src2d, dst2d, z2, z1, *refs):
        if with_deg:
            (out, deg_out, src_v, dst_v, rows_v, acc, sem_i, sem_g, sem_s,
             ones_v, deg_acc, sem_d) = refs
        else:
            out, src_v, dst_v, rows_v, acc, sem_i, sem_g, sem_s = refs
        c = lax.axis_index("c")
        s = lax.axis_index("s")

        pltpu.sync_copy(z2.at[pl.ds(s * T_ROWS, T_ROWS)],
                        acc.at[pl.ds(s * T_ROWS, T_ROWS)])
        if with_deg:
            pltpu.sync_copy(z1.at[pl.ds(s * T_ROWS, T_ROWS)],
                            deg_acc.at[pl.ds(s * T_ROWS, T_ROWS)])
            for i in range(CHUNK // 16):
                ones_v[pl.ds(i * 16, 16)] = jnp.ones((16,), jnp.float32)
        plsc.subcore_barrier()

        tile_id = c * NUM_SUBCORES + s
        row_base = tile_id * ROWS_PER_TILE

        def start_idx(g, slot):
            base = row_base + g * K
            pltpu.async_copy(src2d.at[pl.ds(base, K)], src_v.at[slot], sem_i)
            pltpu.async_copy(dst2d.at[pl.ds(base, K)], dst_v.at[slot], sem_i)

        def drain_idx(slot):
            pltpu.make_async_copy(src2d.at[pl.ds(0, K)], src_v.at[slot],
                                  sem_i).wait()
            pltpu.make_async_copy(dst2d.at[pl.ds(0, K)], dst_v.at[slot],
                                  sem_i).wait()

        def drain_scatters(slot):
            for j in range(K):
                pltpu.make_async_copy(rows_v.at[slot, j],
                                      acc.at[dst_v.at[slot, j]], sem_s).wait()
            if with_deg:
                for j in range(K):
                    pltpu.make_async_copy(
                        ones_v, deg_acc.at[dst_v.at[slot, j]], sem_d).wait()

        start_idx(0, 0)

        def group(g, carry):
            slot = lax.rem(g, 2)
            other = 1 - slot
            drain_idx(slot)
            for j in range(K):
                pltpu.async_copy(table.at[src_v.at[slot, j]],
                                 rows_v.at[slot, j], sem_g)

            @pl.when(g > 0)
            def _():
                drain_scatters(other)

            @pl.when(g + 1 < GROUPS)
            def _():
                start_idx(g + 1, other)

            for j in range(K):
                pltpu.make_async_copy(table.at[src_v.at[slot, j]],
                                      rows_v.at[slot, j], sem_g).wait()
            for j in range(K):
                pltpu.async_copy(rows_v.at[slot, j], acc.at[dst_v.at[slot, j]],
                                 sem_s, add=True)
            if with_deg:
                for j in range(K):
                    pltpu.async_copy(ones_v, deg_acc.at[dst_v.at[slot, j]],
                                     sem_d, add=True)
            return carry

        lax.fori_loop(0, GROUPS, group, 0)
        drain_scatters((GROUPS - 1) % 2)
        plsc.subcore_barrier()

        pltpu.sync_copy(acc.at[pl.ds(s * T_ROWS, T_ROWS)],
                        out.at[c, pl.ds(s * T_ROWS, T_ROWS)])
        if with_deg:
            pltpu.sync_copy(deg_acc.at[pl.ds(s * T_ROWS, T_ROWS)],
                            deg_out.at[c, pl.ds(s * T_ROWS, T_ROWS)])

    return pl.kernel(
        body, out_type=out_type, mesh=mesh, scratch_types=scratch,
        compiler_params=pltpu.CompilerParams(use_tc_tiling_on_sc=False))


BLK = 3584
GRID = N_ACC // BLK


def _tc_merge(parts):
    def body(p_ref, o_ref):
        o_ref[...] = p_ref[0] + p_ref[1]
    return pl.pallas_call(
        body,
        grid=(GRID,),
        in_specs=[pl.BlockSpec((NUM_CORES, BLK, F), lambda i: (0, i, 0))],
        out_specs=pl.BlockSpec((BLK, F), lambda i: (i, 0)),
        out_shape=jax.ShapeDtypeStruct((N_ACC, F), jnp.float32),
    )(parts)


def _tc_final(x_pad, agg1, b_parts, deg_parts, Wrel1, Wroot1, b1, Wrel2,
              Wroot2, b2, fcW, fcb):
    def body(x_ref, a1_ref, bp_ref, dp_ref, wr1_ref, wo1_ref, b1_ref,
             wr2_ref, wo2_ref, b2_ref, fw_ref, fb_ref, o_ref):
        x = x_ref[...]
        a1 = a1_ref[...]
        B = bp_ref[0] + bp_ref[1]
        deg = dp_ref[0] + dp_ref[1]
        dot = functools.partial(jnp.dot, preferred_element_type=jnp.float32)
        h1 = dot(a1, wr1_ref[...].T) + b1_ref[...] + dot(x, wo1_ref[...].T)
        agg2 = (dot(B, wr1_ref[...].T) + dot(a1, wo1_ref[...].T)
                + deg * b1_ref[...])
        h2 = dot(agg2, wr2_ref[...].T) + b2_ref[...] + dot(h1, wo2_ref[...].T)
        o_ref[...] = dot(h2, fw_ref[...].T) + fb_ref[...]

    full = lambda shape: pl.BlockSpec(shape, lambda i: tuple(0 for _ in shape))
    return pl.pallas_call(
        body,
        grid=(GRID,),
        in_specs=[
            pl.BlockSpec((BLK, F), lambda i: (i, 0)),
            pl.BlockSpec((BLK, F), lambda i: (i, 0)),
            pl.BlockSpec((NUM_CORES, BLK, F), lambda i: (0, i, 0)),
            pl.BlockSpec((NUM_CORES, BLK, 1), lambda i: (0, i, 0)),
            full((32, 16)), full((32, 16)), full((32,)),
            full((64, 32)), full((64, 32)), full((64,)),
            full((10, 64)), full((10,)),
        ],
        out_specs=pl.BlockSpec((BLK, 10), lambda i: (i, 0)),
        out_shape=jax.ShapeDtypeStruct((N_ACC, 10), jnp.float32),
    )(x_pad, agg1, b_parts, deg_parts, Wrel1, Wroot1, b1, Wrel2, Wroot2, b2,
      fcW, fcb)


def kernel(x, edge_index, Wrel1, Wroot1, b1, Wrel2, Wroot2, b2, fcW, fcb):
    src = edge_index[0].astype(jnp.int32)
    dst = edge_index[1].astype(jnp.int32)
    P = E_PAD - E
    pad_i = jnp.arange(P, dtype=jnp.int32)
    src_p = jnp.concatenate([src, pad_i % 128]).reshape(E_PAD // CHUNK, CHUNK)
    dst_p = jnp.concatenate([dst, N + (pad_i % 256)]).reshape(
        E_PAD // CHUNK, CHUNK)

    zeros2 = jnp.zeros((N_ACC, F), jnp.float32)
    zeros1 = jnp.zeros((N_ACC,), jnp.float32)
    x_pad = jnp.concatenate([x, jnp.zeros((N_ACC - N, F), jnp.float32)])

    agg1_parts, deg_parts = _sc_pass(True)(x_pad, src_p, dst_p, zeros2, zeros1)
    agg1 = _tc_merge(agg1_parts)
    (b_parts,) = _sc_pass(False)(agg1, src_p, dst_p, zeros2, zeros1)
    out = _tc_final(x_pad, agg1, b_parts,
                    deg_parts.reshape(NUM_CORES, N_ACC, 1), Wrel1, Wroot1, b1,
                    Wrel2, Wroot2, b2, fcW, fcb)
    return out[:N]


# speedup vs baseline: 41.2311x; 1.0420x over previous
import functools

import jax
import jax.numpy as jnp
from jax import lax
from jax.experimental import pallas as pl
from jax.experimental.pallas import tpu as pltpu
from jax.experimental.pallas import tpu_sc as plsc

N = 100000
E = 3200000
F = 16

NUM_CORES = 2
NUM_SUBCORES = 16
NUM_TILES = NUM_CORES * NUM_SUBCORES

CHUNK = 128
K = 4
NCHUNKS = E // CHUNK
BASE_CHUNKS = NCHUNKS // NUM_TILES
EXTRA = NCHUNKS - BASE_CHUNKS * NUM_TILES
MAIN_CHUNKS = (BASE_CHUNKS // K) * K
GROUPS = MAIN_CHUNKS // K
N_ACC = 100352
T_ROWS = N_ACC // NUM_SUBCORES


def _sc_pass(with_deg: bool):
    mesh = plsc.VectorSubcoreMesh(
        core_axis_name="c", subcore_axis_name="s",
        num_cores=NUM_CORES, num_subcores=NUM_SUBCORES)

    out_type = [jax.ShapeDtypeStruct((NUM_CORES, N_ACC, F), jnp.float32)]
    scratch = [
        pltpu.VMEM((2, K, CHUNK), jnp.int32),
        pltpu.VMEM((2, K, CHUNK), jnp.int32),
        pltpu.VMEM((2, K, CHUNK, F), jnp.float32),
        pltpu.VMEM((CHUNK, F), jnp.float32),
        pltpu.VMEM_SHARED((N_ACC, F), jnp.float32),
        pltpu.SemaphoreType.DMA,
        pltpu.SemaphoreType.DMA,
        pltpu.SemaphoreType.DMA,
    ]
    if with_deg:
        out_type.append(jax.ShapeDtypeStruct((NUM_CORES, N_ACC), jnp.float32))
        scratch += [
            pltpu.VMEM((CHUNK,), jnp.float32),
            pltpu.VMEM((CHUNK,), jnp.float32),
            pltpu.VMEM_SHARED((N_ACC,), jnp.float32),
            pltpu.SemaphoreType.DMA,
        ]

    def body(table, src2d, dst2d, *refs):
        if with_deg:
            (out, deg_out, src_v, dst_v, rows_v, zrow, acc, sem_i, sem_g,
             sem_s, ones_v, zone_v, deg_acc, sem_d) = refs
        else:
            out, src_v, dst_v, rows_v, zrow, acc, sem_i, sem_g, sem_s = refs
        c = lax.axis_index("c")
        s = lax.axis_index("s")

        def zfill(i, carry):
            zrow[i] = jnp.zeros((F,), jnp.float32)
            return carry
        lax.fori_loop(0, CHUNK, zfill, 0)
        if with_deg:
            for i in range(CHUNK // 16):
                ones_v[pl.ds(i * 16, 16)] = jnp.ones((16,), jnp.float32)
                zone_v[pl.ds(i * 16, 16)] = jnp.zeros((16,), jnp.float32)
        def zcopy(i, carry):
            base = s * T_ROWS + i * CHUNK
            pltpu.sync_copy(zrow, acc.at[pl.ds(base, CHUNK)])
            if with_deg:
                pltpu.sync_copy(zone_v, deg_acc.at[pl.ds(base, CHUNK)])
            return carry
        lax.fori_loop(0, T_ROWS // CHUNK, zcopy, 0)
        plsc.subcore_barrier()

        t = c * NUM_SUBCORES + s
        start = BASE_CHUNKS * t + jnp.minimum(t, EXTRA)
        n_rem = (BASE_CHUNKS - MAIN_CHUNKS) + jnp.where(t < EXTRA, 1, 0)

        def start_idx(g, slot):
            base = start + g * K
            pltpu.async_copy(src2d.at[pl.ds(base, K)], src_v.at[slot], sem_i)
            pltpu.async_copy(dst2d.at[pl.ds(base, K)], dst_v.at[slot], sem_i)

        def drain_idx(slot):
            pltpu.make_async_copy(src2d.at[pl.ds(0, K)], src_v.at[slot],
                                  sem_i).wait()
            pltpu.make_async_copy(dst2d.at[pl.ds(0, K)], dst_v.at[slot],
                                  sem_i).wait()

        def drain_scatters(slot):
            for j in range(K):
                pltpu.make_async_copy(rows_v.at[slot, j],
                                      acc.at[dst_v.at[slot, j]], sem_s).wait()
            if with_deg:
                for j in range(K):
                    pltpu.make_async_copy(
                        ones_v, deg_acc.at[dst_v.at[slot, j]], sem_d).wait()

        start_idx(0, 0)

        def group(g, carry):
            slot = lax.rem(g, 2)
            other = 1 - slot
            drain_idx(slot)
            for j in range(K):
                pltpu.async_copy(table.at[src_v.at[slot, j]],
                                 rows_v.at[slot, j], sem_g)

            @pl.when(g > 0)
            def _():
                drain_scatters(other)

            @pl.when(g + 1 < GROUPS)
            def _():
                start_idx(g + 1, other)

            for j in range(K):
                pltpu.make_async_copy(table.at[src_v.at[slot, j]],
                                      rows_v.at[slot, j], sem_g).wait()
            for j in range(K):
                pltpu.async_copy(rows_v.at[slot, j], acc.at[dst_v.at[slot, j]],
                                 sem_s, add=True)
            if with_deg:
                for j in range(K):
                    pltpu.async_copy(ones_v, deg_acc.at[dst_v.at[slot, j]],
                                     sem_d, add=True)
            return carry

        lax.fori_loop(0, GROUPS, group, 0)
        drain_scatters((GROUPS - 1) % 2)

        def rem_chunk(r, carry):
            base = start + MAIN_CHUNKS + r
            pltpu.sync_copy(src2d.at[pl.ds(base, 1)], src_v.at[0, pl.ds(0, 1)])
            pltpu.sync_copy(dst2d.at[pl.ds(base, 1)], dst_v.at[0, pl.ds(0, 1)])
            pltpu.async_copy(table.at[src_v.at[0, 0]], rows_v.at[0, 0],
                             sem_g).wait()
            pltpu.sync_copy(rows_v.at[0, 0], acc.at[dst_v.at[0, 0]], add=True)
            if with_deg:
                pltpu.sync_copy(ones_v, deg_acc.at[dst_v.at[0, 0]], add=True)
            return carry
        lax.fori_loop(0, n_rem, rem_chunk, 0)
        plsc.subcore_barrier()

        pltpu.sync_copy(acc.at[pl.ds(s * T_ROWS, T_ROWS)],
                        out.at[c, pl.ds(s * T_ROWS, T_ROWS)])
        if with_deg:
            pltpu.sync_copy(deg_acc.at[pl.ds(s * T_ROWS, T_ROWS)],
                            deg_out.at[c, pl.ds(s * T_ROWS, T_ROWS)])

    return pl.kernel(
        body, out_type=out_type, mesh=mesh, scratch_types=scratch,
        compiler_params=pltpu.CompilerParams(use_tc_tiling_on_sc=False))


MBLK = 3584
BLK = 5000


def _tc_merge(parts):
    def body(p_ref, o_ref):
        o_ref[...] = p_ref[0] + p_ref[1]
    return pl.pallas_call(
        body,
        grid=(N_ACC // MBLK,),
        in_specs=[pl.BlockSpec((NUM_CORES, MBLK, F), lambda i: (0, i, 0))],
        out_specs=pl.BlockSpec((MBLK, F), lambda i: (i, 0)),
        out_shape=jax.ShapeDtypeStruct((N_ACC, F), jnp.float32),
    )(parts)


def _tc_final(x, agg1, b_parts, deg_parts, Wrel1, Wroot1, b1, Wrel2,
              Wroot2, b2, fcW, fcb):
    def body(x_ref, a1_ref, bp_ref, dp_ref, wr1_ref, wo1_ref, b1_ref,
             wr2_ref, wo2_ref, b2_ref, fw_ref, fb_ref, o_ref):
        x_ = x_ref[...]
        a1 = a1_ref[...]
        B = bp_ref[0] + bp_ref[1]
        deg = dp_ref[0] + dp_ref[1]
        dot = functools.partial(jnp.dot, preferred_element_type=jnp.float32)
        h1 = dot(a1, wr1_ref[...].T) + b1_ref[...] + dot(x_, wo1_ref[...].T)
        agg2 = (dot(B, wr1_ref[...].T) + dot(a1, wo1_ref[...].T)
                + deg * b1_ref[...])
        h2 = dot(agg2, wr2_ref[...].T) + b2_ref[...] + dot(h1, wo2_ref[...].T)
        o_ref[...] = dot(h2, fw_ref[...].T) + fb_ref[...]

    full = lambda shape: pl.BlockSpec(shape, lambda i: tuple(0 for _ in shape))
    return pl.pallas_call(
        body,
        grid=(N // BLK,),
        in_specs=[
            pl.BlockSpec((BLK, F), lambda i: (i, 0)),
            pl.BlockSpec((BLK, F), lambda i: (i, 0)),
            pl.BlockSpec((NUM_CORES, BLK, F), lambda i: (0, i, 0)),
            pl.BlockSpec((NUM_CORES, BLK, 1), lambda i: (0, i, 0)),
            full((32, 16)), full((32, 16)), full((32,)),
            full((64, 32)), full((64, 32)), full((64,)),
            full((10, 64)), full((10,)),
        ],
        out_specs=pl.BlockSpec((BLK, 10), lambda i: (i, 0)),
        out_shape=jax.ShapeDtypeStruct((N, 10), jnp.float32),
    )(x, agg1, b_parts, deg_parts, Wrel1, Wroot1, b1, Wrel2, Wroot2, b2,
      fcW, fcb)


def kernel(x, edge_index, Wrel1, Wroot1, b1, Wrel2, Wroot2, b2, fcW, fcb):
    src2d = edge_index[0].astype(jnp.int32).reshape(NCHUNKS, CHUNK)
    dst2d = edge_index[1].astype(jnp.int32).reshape(NCHUNKS, CHUNK)

    agg1_parts, deg_parts = _sc_pass(True)(x, src2d, dst2d)
    agg1 = _tc_merge(agg1_parts)
    (b_parts,) = _sc_pass(False)(agg1, src2d, dst2d)
    return _tc_final(x, agg1, b_parts, deg_parts.reshape(NUM_CORES, N_ACC, 1),
                     Wrel1, Wroot1, b1, Wrel2, Wroot2, b2, fcW, fcb)


# speedup vs baseline: 51.0370x; 1.2378x over previous
import functools

import jax
import jax.numpy as jnp
from jax import lax
from jax.experimental import pallas as pl
from jax.experimental.pallas import tpu as pltpu
from jax.experimental.pallas import tpu_sc as plsc

N = 100000
E = 3200000
F = 16

NUM_CORES = 2
NUM_SUBCORES = 16
NUM_TILES = NUM_CORES * NUM_SUBCORES

CHUNK = 128
K = 4
NCHUNKS = E // CHUNK
BASE_CHUNKS = NCHUNKS // NUM_TILES
EXTRA = NCHUNKS - BASE_CHUNKS * NUM_TILES
MAIN_CHUNKS = (BASE_CHUNKS // K) * K
GROUPS = MAIN_CHUNKS // K
N_ACC = 100352
T_ROWS = N_ACC // NUM_SUBCORES
M_ROWS = N_ACC // NUM_TILES


def _sc_pass(with_deg: bool):
    mesh = plsc.VectorSubcoreMesh(
        core_axis_name="c", subcore_axis_name="s",
        num_cores=NUM_CORES, num_subcores=NUM_SUBCORES)

    out_type = [jax.ShapeDtypeStruct((NUM_CORES, N_ACC, F), jnp.float32)]
    scratch = [
        pltpu.VMEM((2, K, CHUNK), jnp.int32),
        pltpu.VMEM((2, K, CHUNK), jnp.int32),
        pltpu.VMEM((2, K, CHUNK, F), jnp.float32),
        pltpu.VMEM((CHUNK, F), jnp.float32),
        pltpu.VMEM_SHARED((N_ACC, F), jnp.float32),
        pltpu.SemaphoreType.DMA,
        pltpu.SemaphoreType.DMA,
        pltpu.SemaphoreType.DMA,
    ]
    if with_deg:
        out_type.append(jax.ShapeDtypeStruct((NUM_CORES, N_ACC), jnp.float32))
        scratch += [
            pltpu.VMEM((CHUNK,), jnp.float32),
            pltpu.VMEM((CHUNK,), jnp.float32),
            pltpu.VMEM_SHARED((N_ACC,), jnp.float32),
            pltpu.SemaphoreType.DMA,
        ]

    def body(table, edges, *refs):
        if with_deg:
            (out, deg_out, src_v, dst_v, rows_v, zrow, acc, sem_i, sem_g,
             sem_s, ones_v, zone_v, deg_acc, sem_d) = refs
        else:
            out, src_v, dst_v, rows_v, zrow, acc, sem_i, sem_g, sem_s = refs
        c = lax.axis_index("c")
        s = lax.axis_index("s")

        def zfill(i, carry):
            zrow[i] = jnp.zeros((F,), jnp.float32)
            return carry
        lax.fori_loop(0, CHUNK, zfill, 0)
        if with_deg:
            for i in range(CHUNK // 16):
                ones_v[pl.ds(i * 16, 16)] = jnp.ones((16,), jnp.float32)
                zone_v[pl.ds(i * 16, 16)] = jnp.zeros((16,), jnp.float32)
        def zcopy(i, carry):
            base = s * T_ROWS + i * CHUNK
            pltpu.sync_copy(zrow, acc.at[pl.ds(base, CHUNK)])
            if with_deg:
                pltpu.sync_copy(zone_v, deg_acc.at[pl.ds(base, CHUNK)])
            return carry
        lax.fori_loop(0, T_ROWS // CHUNK, zcopy, 0)
        plsc.subcore_barrier()

        t = c * NUM_SUBCORES + s
        start = BASE_CHUNKS * t + jnp.minimum(t, EXTRA)
        n_rem = (BASE_CHUNKS - MAIN_CHUNKS) + jnp.where(t < EXTRA, 1, 0)

        def start_idx(g, slot):
            base = start + g * K
            pltpu.async_copy(edges.at[0, pl.ds(base, K)], src_v.at[slot],
                             sem_i)
            pltpu.async_copy(edges.at[1, pl.ds(base, K)], dst_v.at[slot],
                             sem_i)

        def drain_idx(slot):
            pltpu.make_async_copy(edges.at[0, pl.ds(0, K)], src_v.at[slot],
                                  sem_i).wait()
            pltpu.make_async_copy(edges.at[1, pl.ds(0, K)], dst_v.at[slot],
                                  sem_i).wait()

        def drain_scatters(slot):
            for j in range(K):
                pltpu.make_async_copy(rows_v.at[slot, j],
                                      acc.at[dst_v.at[slot, j]], sem_s).wait()
            if with_deg:
                for j in range(K):
                    pltpu.make_async_copy(
                        ones_v, deg_acc.at[dst_v.at[slot, j]], sem_d).wait()

        start_idx(0, 0)

        def group(g, carry):
            slot = lax.rem(g, 2)
            other = 1 - slot
            drain_idx(slot)
            for j in range(K):
                pltpu.async_copy(table.at[src_v.at[slot, j]],
                                 rows_v.at[slot, j], sem_g)

            @pl.when(g > 0)
            def _():
                drain_scatters(other)

            @pl.when(g + 1 < GROUPS)
            def _():
                start_idx(g + 1, other)

            for j in range(K):
                pltpu.make_async_copy(table.at[src_v.at[slot, j]],
                                      rows_v.at[slot, j], sem_g).wait()
            for j in range(K):
                pltpu.async_copy(rows_v.at[slot, j], acc.at[dst_v.at[slot, j]],
                                 sem_s, add=True)
            if with_deg:
                for j in range(K):
                    pltpu.async_copy(ones_v, deg_acc.at[dst_v.at[slot, j]],
                                     sem_d, add=True)
            return carry

        lax.fori_loop(0, GROUPS, group, 0)
        drain_scatters((GROUPS - 1) % 2)

        def rem_chunk(r, carry):
            base = start + MAIN_CHUNKS + r
            pltpu.sync_copy(edges.at[0, pl.ds(base, 1)],
                            src_v.at[0, pl.ds(0, 1)])
            pltpu.sync_copy(edges.at[1, pl.ds(base, 1)],
                            dst_v.at[0, pl.ds(0, 1)])
            pltpu.async_copy(table.at[src_v.at[0, 0]], rows_v.at[0, 0],
                             sem_g).wait()
            pltpu.sync_copy(rows_v.at[0, 0], acc.at[dst_v.at[0, 0]], add=True)
            if with_deg:
                pltpu.sync_copy(ones_v, deg_acc.at[dst_v.at[0, 0]], add=True)
            return carry
        lax.fori_loop(0, n_rem, rem_chunk, 0)
        plsc.subcore_barrier()

        pltpu.sync_copy(acc.at[pl.ds(s * T_ROWS, T_ROWS)],
                        out.at[c, pl.ds(s * T_ROWS, T_ROWS)])
        if with_deg:
            pltpu.sync_copy(deg_acc.at[pl.ds(s * T_ROWS, T_ROWS)],
                            deg_out.at[c, pl.ds(s * T_ROWS, T_ROWS)])

    return pl.kernel(
        body, out_type=out_type, mesh=mesh, scratch_types=scratch,
        compiler_params=pltpu.CompilerParams(use_tc_tiling_on_sc=False))


def _sc_merge():
    mesh = plsc.VectorSubcoreMesh(
        core_axis_name="c", subcore_axis_name="s",
        num_cores=NUM_CORES, num_subcores=NUM_SUBCORES)
    scratch = [
        pltpu.VMEM((M_ROWS, F), jnp.float32),
        pltpu.VMEM((M_ROWS, F), jnp.float32),
    ]

    def body(parts, out, buf0, buf1):
        c = lax.axis_index("c")
        s = lax.axis_index("s")
        t = c * NUM_SUBCORES + s
        base = t * M_ROWS
        pltpu.sync_copy(parts.at[0, pl.ds(base, M_ROWS)], buf0)
        pltpu.sync_copy(parts.at[1, pl.ds(base, M_ROWS)], buf1)

        def add4(i, carry):
            for u in range(4):
                r = i * 4 + u
                buf0[r] = buf0[r] + buf1[r]
            return carry
        lax.fori_loop(0, M_ROWS // 4, add4, 0)
        pltpu.sync_copy(buf0, out.at[pl.ds(base, M_ROWS)])

    return pl.kernel(
        body, out_type=jax.ShapeDtypeStruct((N_ACC, F), jnp.float32),
        mesh=mesh, scratch_types=scratch,
        compiler_params=pltpu.CompilerParams(use_tc_tiling_on_sc=False))


BLK = 5120


def _tc_final(x, agg1, b_parts, deg_parts, Wrel1, Wroot1, b1, Wrel2,
              Wroot2, b2, fcW, fcb):
    def body(x_ref, a1_ref, bp_ref, dp_ref, wr1_ref, wo1_ref, b1_ref,
             wr2_ref, wo2_ref, b2_ref, fw_ref, fb_ref, o_ref):
        dot = functools.partial(jnp.dot, preferred_element_type=jnp.float32)
        wr1t = wr1_ref[...].T
        wo1t = wo1_ref[...].T
        w2f = dot(wr2_ref[...].T, fw_ref[...].T)
        wo2f = dot(wo2_ref[...].T, fw_ref[...].T)
        M1 = dot(wo1t, wo2f)
        M2 = dot(wo1t, w2f) + dot(wr1t, wo2f)
        M3 = dot(wr1t, w2f)
        b1r = b1_ref[...].reshape(1, 32)
        b2r = b2_ref[...].reshape(1, 64)
        v = dot(b1r, w2f)
        const = (dot(dot(b1r, wo2_ref[...].T) + b2r, fw_ref[...].T)
                 + fb_ref[...].reshape(1, 10))
        B = bp_ref[0] + bp_ref[1]
        dp = dp_ref[...]
        deg_term = lax.dot_general(
            dp, jnp.broadcast_to(v, (2, 10)), (((0,), (0,)), ((), ())),
            preferred_element_type=jnp.float32)
        o_ref[...] = (dot(B, M3) + dot(a1_ref[...], M2) + dot(x_ref[...], M1)
                      + deg_term + const)

    full = lambda shape: pl.BlockSpec(shape, lambda i: tuple(0 for _ in shape))
    return pl.pallas_call(
        body,
        grid=(pl.cdiv(N, BLK),),
        in_specs=[
            pl.BlockSpec((BLK, F), lambda i: (i, 0)),
            pl.BlockSpec((BLK, F), lambda i: (i, 0)),
            pl.BlockSpec((NUM_CORES, BLK, F), lambda i: (0, i, 0)),
            pl.BlockSpec((NUM_CORES, BLK), lambda i: (0, i)),
            full((32, 16)), full((32, 16)), full((32,)),
            full((64, 32)), full((64, 32)), full((64,)),
            full((10, 64)), full((10,)),
        ],
        out_specs=pl.BlockSpec((BLK, 10), lambda i: (i, 0)),
        out_shape=jax.ShapeDtypeStruct((N, 10), jnp.float32),
    )(x, agg1, b_parts, deg_parts, Wrel1, Wroot1, b1, Wrel2, Wroot2, b2,
      fcW, fcb)


def kernel(x, edge_index, Wrel1, Wroot1, b1, Wrel2, Wroot2, b2, fcW, fcb):
    edges = edge_index.astype(jnp.int32).reshape(2, NCHUNKS, CHUNK)
    agg1_parts, deg_parts = _sc_pass(True)(x, edges)
    agg1 = _sc_merge()(agg1_parts)
    (b_parts,) = _sc_pass(False)(agg1, edges)
    return _tc_final(x, agg1, b_parts, deg_parts, Wrel1, Wroot1, b1,
                     Wrel2, Wroot2, b2, fcW, fcb)


# speedup vs baseline: 55.4358x; 1.0862x over previous
import functools

import jax
import jax.numpy as jnp
from jax import lax
from jax.experimental import pallas as pl
from jax.experimental.pallas import tpu as pltpu
from jax.experimental.pallas import tpu_sc as plsc

N = 100000
E = 3200000
F = 16

NUM_CORES = 2
NUM_SUBCORES = 16
NUM_TILES = NUM_CORES * NUM_SUBCORES

CHUNK = 128
K = 4
NCHUNKS = E // CHUNK
BASE_CHUNKS = NCHUNKS // NUM_TILES
EXTRA = NCHUNKS - BASE_CHUNKS * NUM_TILES
MAIN_CHUNKS = (BASE_CHUNKS // K) * K
GROUPS = MAIN_CHUNKS // K
N_ACC = 100352
T_ROWS = N_ACC // NUM_SUBCORES
M_ROWS = N_ACC // NUM_TILES


def _sc_pass(with_deg: bool):
    mesh = plsc.VectorSubcoreMesh(
        core_axis_name="c", subcore_axis_name="s",
        num_cores=NUM_CORES, num_subcores=NUM_SUBCORES)

    out_type = [jax.ShapeDtypeStruct((NUM_CORES, N_ACC, F), jnp.float32)]
    scratch = [
        pltpu.VMEM((2, K, CHUNK), jnp.int32),
        pltpu.VMEM((2, K, CHUNK), jnp.int32),
        pltpu.VMEM((2, K, CHUNK, F), jnp.float32),
        pltpu.VMEM((CHUNK, F), jnp.float32),
        pltpu.VMEM_SHARED((N_ACC, F), jnp.float32),
        pltpu.SemaphoreType.DMA,
        pltpu.SemaphoreType.DMA,
        pltpu.SemaphoreType.DMA,
    ]
    if with_deg:
        out_type.append(jax.ShapeDtypeStruct((NUM_CORES, N_ACC), jnp.float32))
        scratch += [
            pltpu.VMEM((CHUNK,), jnp.float32),
            pltpu.VMEM((CHUNK,), jnp.float32),
            pltpu.VMEM_SHARED((N_ACC,), jnp.float32),
            pltpu.SemaphoreType.DMA,
        ]

    def body(table, edges, *refs):
        if with_deg:
            (out, deg_out, src_v, dst_v, rows_v, zrow, acc, sem_i, sem_g,
             sem_s, ones_v, zone_v, deg_acc, sem_d) = refs
        else:
            out, src_v, dst_v, rows_v, zrow, acc, sem_i, sem_g, sem_s = refs
        c = lax.axis_index("c")
        s = lax.axis_index("s")

        def zfill(i, carry):
            zrow[i] = jnp.zeros((F,), jnp.float32)
            return carry
        lax.fori_loop(0, CHUNK, zfill, 0)
        if with_deg:
            for i in range(CHUNK // 16):
                ones_v[pl.ds(i * 16, 16)] = jnp.ones((16,), jnp.float32)
                zone_v[pl.ds(i * 16, 16)] = jnp.zeros((16,), jnp.float32)
        def zcopy(i, carry):
            base = s * T_ROWS + i * CHUNK
            pltpu.sync_copy(zrow, acc.at[pl.ds(base, CHUNK)])
            if with_deg:
                pltpu.sync_copy(zone_v, deg_acc.at[pl.ds(base, CHUNK)])
            return carry
        lax.fori_loop(0, T_ROWS // CHUNK, zcopy, 0)
        plsc.subcore_barrier()

        t = c * NUM_SUBCORES + s
        start = BASE_CHUNKS * t + jnp.minimum(t, EXTRA)
        n_rem = (BASE_CHUNKS - MAIN_CHUNKS) + jnp.where(t < EXTRA, 1, 0)

        def start_idx(g, slot):
            base = start + g * K
            pltpu.async_copy(edges.at[0, pl.ds(base, K)], src_v.at[slot],
                             sem_i)
            pltpu.async_copy(edges.at[1, pl.ds(base, K)], dst_v.at[slot],
                             sem_i)

        def drain_idx(slot):
            pltpu.make_async_copy(edges.at[0, pl.ds(0, K)], src_v.at[slot],
                                  sem_i).wait()
            pltpu.make_async_copy(edges.at[1, pl.ds(0, K)], dst_v.at[slot],
                                  sem_i).wait()

        def drain_scatters(slot):
            for j in range(K):
                pltpu.make_async_copy(rows_v.at[slot, j],
                                      acc.at[dst_v.at[slot, j]], sem_s).wait()
            if with_deg:
                for j in range(K):
                    pltpu.make_async_copy(
                        ones_v, deg_acc.at[dst_v.at[slot, j]], sem_d).wait()

        start_idx(0, 0)

        def group(g, carry):
            slot = lax.rem(g, 2)
            other = 1 - slot
            drain_idx(slot)
            for j in range(K):
                pltpu.async_copy(table.at[src_v.at[slot, j]],
                                 rows_v.at[slot, j], sem_g)

            @pl.when(g > 0)
            def _():
                drain_scatters(other)

            @pl.when(g + 1 < GROUPS)
            def _():
                start_idx(g + 1, other)

            for j in range(K):
                pltpu.make_async_copy(table.at[src_v.at[slot, j]],
                                      rows_v.at[slot, j], sem_g).wait()
            for j in range(K):
                pltpu.async_copy(rows_v.at[slot, j], acc.at[dst_v.at[slot, j]],
                                 sem_s, add=True)
            if with_deg:
                for j in range(K):
                    pltpu.async_copy(ones_v, deg_acc.at[dst_v.at[slot, j]],
                                     sem_d, add=True)
            return carry

        lax.fori_loop(0, GROUPS, group, 0)
        drain_scatters((GROUPS - 1) % 2)

        def rem_chunk(r, carry):
            base = start + MAIN_CHUNKS + r
            pltpu.sync_copy(edges.at[0, pl.ds(base, 1)],
                            src_v.at[0, pl.ds(0, 1)])
            pltpu.sync_copy(edges.at[1, pl.ds(base, 1)],
                            dst_v.at[0, pl.ds(0, 1)])
            pltpu.async_copy(table.at[src_v.at[0, 0]], rows_v.at[0, 0],
                             sem_g).wait()
            pltpu.sync_copy(rows_v.at[0, 0], acc.at[dst_v.at[0, 0]], add=True)
            if with_deg:
                pltpu.sync_copy(ones_v, deg_acc.at[dst_v.at[0, 0]], add=True)
            return carry
        lax.fori_loop(0, n_rem, rem_chunk, 0)
        plsc.subcore_barrier()

        pltpu.sync_copy(acc.at[pl.ds(s * T_ROWS, T_ROWS)],
                        out.at[c, pl.ds(s * T_ROWS, T_ROWS)])
        if with_deg:
            pltpu.sync_copy(deg_acc.at[pl.ds(s * T_ROWS, T_ROWS)],
                            deg_out.at[c, pl.ds(s * T_ROWS, T_ROWS)])

    return pl.kernel(
        body, out_type=out_type, mesh=mesh, scratch_types=scratch,
        compiler_params=pltpu.CompilerParams(use_tc_tiling_on_sc=False))


def _sc_merge(with_deg: bool):
    mesh = plsc.VectorSubcoreMesh(
        core_axis_name="c", subcore_axis_name="s",
        num_cores=NUM_CORES, num_subcores=NUM_SUBCORES)
    out_type = [jax.ShapeDtypeStruct((N_ACC, F), jnp.float32)]
    scratch = [
        pltpu.VMEM((M_ROWS, F), jnp.float32),
        pltpu.VMEM((M_ROWS, F), jnp.float32),
    ]
    if with_deg:
        out_type.append(jax.ShapeDtypeStruct((N_ACC,), jnp.float32))
        scratch += [
            pltpu.VMEM((M_ROWS,), jnp.float32),
            pltpu.VMEM((M_ROWS,), jnp.float32),
        ]

    def body(*refs):
        if with_deg:
            parts, degp, out, deg_out, buf0, buf1, db0, db1 = refs
        else:
            parts, out, buf0, buf1 = refs
        c = lax.axis_index("c")
        s = lax.axis_index("s")
        t = c * NUM_SUBCORES + s
        base = t * M_ROWS
        pltpu.sync_copy(parts.at[0, pl.ds(base, M_ROWS)], buf0)
        pltpu.sync_copy(parts.at[1, pl.ds(base, M_ROWS)], buf1)
        if with_deg:
            pltpu.sync_copy(degp.at[0, pl.ds(base, M_ROWS)], db0)
            pltpu.sync_copy(degp.at[1, pl.ds(base, M_ROWS)], db1)

        def add4(i, carry):
            for u in range(4):
                r = i * 4 + u
                buf0[r] = buf0[r] + buf1[r]
            return carry
        lax.fori_loop(0, M_ROWS // 4, add4, 0)
        pltpu.sync_copy(buf0, out.at[pl.ds(base, M_ROWS)])
        if with_deg:
            def dadd(i, carry):
                sl = pl.ds(i * 16, 16)
                db0[sl] = db0[sl] + db1[sl]
                return carry
            lax.fori_loop(0, M_ROWS // 16, dadd, 0)
            pltpu.sync_copy(db0, deg_out.at[pl.ds(base, M_ROWS)])

    return pl.kernel(
        body, out_type=out_type, mesh=mesh, scratch_types=scratch,
        compiler_params=pltpu.CompilerParams(use_tc_tiling_on_sc=False))


R_PACK = N // 8
R_ACC = N_ACC // 8
FBLK = 640


def _tc_final(x128, a128, b128, deg8, Wrel1, Wroot1, b1, Wrel2, Wroot2, b2,
              fcW, fcb):
    def body(x_ref, a1_ref, b_ref, dp_ref, wr1_ref, wo1_ref, b1_ref,
             wr2_ref, wo2_ref, b2_ref, fw_ref, fb_ref, o_ref):
        dot = functools.partial(jnp.dot, preferred_element_type=jnp.float32)
        wr1t = wr1_ref[...].T
        wo1t = wo1_ref[...].T
        w2f = dot(wr2_ref[...].T, fw_ref[...].T)
        wo2f = dot(wo2_ref[...].T, fw_ref[...].T)
        M1 = dot(wo1t, wo2f)
        M2 = dot(wo1t, w2f) + dot(wr1t, wo2f)
        M3 = dot(wr1t, w2f)
        b1r = b1_ref[...].reshape(1, 32)
        b2r = b2_ref[...].reshape(1, 64)
        v = dot(b1r, w2f)
        const = (dot(dot(b1r, wo2_ref[...].T) + b2r, fw_ref[...].T)
                 + fb_ref[...].reshape(1, 10))

        def kron8(M, nr, nc):
            Mt = jnp.tile(M, (8, 8))
            rb = lax.broadcasted_iota(jnp.int32, (8 * nr, 8 * nc), 0) // nr
            cb = lax.broadcasted_iota(jnp.int32, (8 * nr, 8 * nc), 1) // nc
            return jnp.where(rb == cb, Mt, jnp.float32(0))

        W1 = kron8(M1, 16, 10)
        W2 = kron8(M2, 16, 10)
        W3 = kron8(M3, 16, 10)
        V8 = kron8(v, 1, 10)
        o_ref[...] = (dot(b_ref[...], W3) + dot(a1_ref[...], W2)
                      + dot(x_ref[...], W1) + dot(dp_ref[...], V8)
                      + jnp.tile(const, (1, 8)))

    full = lambda shape: pl.BlockSpec(shape, lambda i: tuple(0 for _ in shape))
    return pl.pallas_call(
        body,
        grid=(pl.cdiv(R_PACK, FBLK),),
        in_specs=[
            pl.BlockSpec((FBLK, 128), lambda i: (i, 0)),
            pl.BlockSpec((FBLK, 128), lambda i: (i, 0)),
            pl.BlockSpec((FBLK, 128), lambda i: (i, 0)),
            pl.BlockSpec((FBLK, 8), lambda i: (i, 0)),
            full((32, 16)), full((32, 16)), full((32,)),
            full((64, 32)), full((64, 32)), full((64,)),
            full((10, 64)), full((10,)),
        ],
        out_specs=pl.BlockSpec((FBLK, 80), lambda i: (i, 0)),
        out_shape=jax.ShapeDtypeStruct((R_PACK, 80), jnp.float32),
    )(x128, a128, b128, deg8, Wrel1, Wroot1, b1, Wrel2, Wroot2, b2, fcW, fcb)


def kernel(x, edge_index, Wrel1, Wroot1, b1, Wrel2, Wroot2, b2, fcW, fcb):
    edges = edge_index.astype(jnp.int32).reshape(2, NCHUNKS, CHUNK)
    x128 = x.reshape(R_PACK, 128)
    x_sc = x128.reshape(N, F)
    agg1_parts, deg_parts = _sc_pass(True)(x_sc, edges)
    agg1, deg = _sc_merge(True)(agg1_parts, deg_parts)
    (b_parts,) = _sc_pass(False)(agg1, edges)
    (b_sum,) = _sc_merge(False)(b_parts)
    out = _tc_final(x128, agg1.reshape(R_ACC, 128), b_sum.reshape(R_ACC, 128),
                    deg.reshape(R_ACC, 8), Wrel1, Wroot1, b1, Wrel2, Wroot2,
                    b2, fcW, fcb)
    return out.reshape(N, 10)
